# Initial kernel scaffold; baseline (speedup 1.0000x reference)
import jax
import jax.numpy as jnp
from jax.experimental import pallas as pl


def kernel(x, table):
    raise NotImplementedError("write your pallas kernel here")


# speedup vs baseline: 79.5681x; 79.5681x over previous
import functools

import jax
import jax.numpy as jnp
import numpy as np
from jax import lax
from jax.experimental import pallas as pl
from jax.experimental.pallas import tpu as pltpu
from jax.experimental.pallas import tpu_sc as plsc

_N_LEVELS = 16
_F = 2
_LOG2_T = 19
_T = 1 << _LOG2_T
_N = 262144
_P2 = np.int32(np.uint32(2654435761).view(np.int32))
_P3 = np.int32(805459861)

_NC = 2
_NS = 16
_NW = _NC * _NS
_PW = _N // _NW
_C = 2048
_NCHUNK = _PW // _C
_G = _C // 16
_DENSE_LEVELS = 3


def _splat(v, dtype=jnp.int32):
    return jnp.full((16,), v, dtype=dtype)


def _body(x_hbm, tab0_hbm, tab1_hbm, out_hbm, x_v, idx_v, feat0_v, feat1_v, out_v, sem):
    cid = lax.axis_index("c")
    sid = lax.axis_index("s")
    wid = sid * _NC + cid
    iota = lax.iota(jnp.int32, 16)

    def load_pos(g, scale):
        px = x_v[pl.ds(g * 16, 16)]
        py = x_v[pl.ds(_C + g * 16, 16)]
        pz = x_v[pl.ds(2 * _C + g * 16, 16)]
        posx = px * scale + 0.5
        posy = py * scale + 0.5
        posz = pz * scale + 0.5
        ix = posx.astype(jnp.int32)
        iy = posy.astype(jnp.int32)
        iz = posz.astype(jnp.int32)
        wx = posx - ix.astype(jnp.float32)
        wy = posy - iy.astype(jnp.float32)
        wz = posz - iz.astype(jnp.float32)
        return ix, iy, iz, wx, wy, wz

    def store_idx(c, g, idx):
        idx_v[pl.ds(c * _C + g * 16, 16)] = idx

    def idx_pass(g, scale, lvl_base, dense, r1):
        ix, iy, iz, _, _, _ = load_pos(g, scale)
        if dense:
            ay0 = iy * r1
            ay1 = ay0 + r1
            r1sq = r1 * r1
            az0 = iz * r1sq
            az1 = az0 + r1sq
            ax0, ax1 = ix, ix + 1
            for c in range(8):
                ax = ax1 if (c >> 2) & 1 else ax0
                ay = ay1 if (c >> 1) & 1 else ay0
                az = az1 if c & 1 else az0
                store_idx(c, g, ax + ay + az + lvl_base)
        else:
            by0 = iy * _P2
            by1 = by0 + _P2
            bz0 = iz * _P3
            bz1 = bz0 + _P3
            ax0, ax1 = ix, ix + 1
            for c in range(8):
                ax = ax1 if (c >> 2) & 1 else ax0
                by = by1 if (c >> 1) & 1 else by0
                bz = bz1 if c & 1 else bz0
                h = (ax ^ by ^ bz) & (_T - 1)
                store_idx(c, g, h + lvl_base)

    def acc_pass(g, scale):
        _, _, _, wx, wy, wz = load_pos(g, scale)
        ux, uy, uz = 1.0 - wx, 1.0 - wy, 1.0 - wz
        w00 = ux * uy
        w01 = ux * wy
        w10 = wx * uy
        w11 = wx * wy
        wxy = (w00, w01, w10, w11)
        acc0 = jnp.zeros((16,), jnp.float32)
        acc1 = jnp.zeros((16,), jnp.float32)
        for c in range(8):
            wc = wxy[c >> 1] * (wz if c & 1 else uz)
            f0 = feat0_v[pl.ds(c * _C + g * 16, 16)]
            f1 = feat1_v[pl.ds(c * _C + g * 16, 16)]
            acc0 = acc0 + wc * f0
            acc1 = acc1 + wc * f1
        out_v[pl.ds(g * 16, 16)] = acc0
        out_v[pl.ds(_C + g * 16, 16)] = acc1

    def do_level(l, scale, cbase, dense, r1):
        def idx_g(g, carry):
            idx_pass(g, scale, l * _T, dense, r1)
            return carry

        lax.fori_loop(0, _G, idx_g, 0)
        cp0 = pltpu.async_copy(tab0_hbm.at[idx_v], feat0_v, sem)
        cp1 = pltpu.async_copy(tab1_hbm.at[idx_v], feat1_v, sem)
        cp0.wait()
        cp1.wait()

        def acc_g(g, carry):
            acc_pass(g, scale)
            return carry

        lax.fori_loop(0, _G, acc_g, 0)
        lf = l * (2 * _N)
        pltpu.sync_copy(out_v.at[pl.ds(0, _C)], out_hbm.at[pl.ds(lf + cbase, _C)])
        pltpu.sync_copy(out_v.at[pl.ds(_C, _C)], out_hbm.at[pl.ds(lf + _N + cbase, _C)])

    def chunk_body(ci, carry):
        cbase = wid * _PW + ci * _C
        for d in range(3):
            pltpu.sync_copy(x_hbm.at[pl.ds(d * _N + cbase, _C)],
                            x_v.at[pl.ds(d * _C, _C)])
        for l in range(_DENSE_LEVELS):
            res = 16 << l
            do_level(l, float(res - 1), cbase, True, res + 1)

        def hash_level(l, c2):
            s_i = lax.shift_left(jnp.int32(1), l + 4)
            scale = s_i.astype(jnp.float32) - 1.0
            do_level(l, scale, cbase, False, 0)
            return c2

        lax.fori_loop(_DENSE_LEVELS, _N_LEVELS, hash_level, 0)
        return carry

    lax.fori_loop(0, _NCHUNK, chunk_body, 0)


_mesh = plsc.VectorSubcoreMesh(
    core_axis_name="c", subcore_axis_name="s", num_cores=_NC, num_subcores=_NS
)

_embed = pl.kernel(
    _body,
    out_type=jax.ShapeDtypeStruct((_N_LEVELS * _F * _N,), jnp.float32),
    mesh=_mesh,
    scratch_types=[
        pltpu.VMEM((3 * _C,), jnp.float32),
        pltpu.VMEM((8 * _C,), jnp.int32),
        pltpu.VMEM((8 * _C,), jnp.float32),
        pltpu.VMEM((8 * _C,), jnp.float32),
        pltpu.VMEM((_F * _C,), jnp.float32),
        pltpu.SemaphoreType.DMA,
    ],
)


def kernel(x, table):
    x_t = x.T.reshape(3 * _N)
    tab = table.reshape(_N_LEVELS * _T, _F)
    out = _embed(x_t, tab[:, 0], tab[:, 1]).reshape(_N_LEVELS, _F, _N)
    return out.transpose(2, 0, 1).reshape(_N, _N_LEVELS * _F)
